# baseline (speedup 1.0000x reference)
import jax
import jax.numpy as jnp
from jax.experimental import pallas as pl
from jax.experimental.pallas import tpu as pltpu

DIM = 1024
NUM_EXPERTS = 8
N_ROUTED = 7
HIDDEN = 4096
ALPHA = 2.0
TOKENS = 64
HB = 1024
NH = HIDDEN // HB


def _dotT(a, b):
    return jax.lax.dot_general(a, b, (((1,), (1,)), ((), ())),
                               preferred_element_type=jnp.float32)


def _swiglu_part(x, wu, wg, wd):
    up = _dotT(x, wu)
    g = _dotT(x, wg)
    act = up * (g * jax.nn.sigmoid(g))
    return _dotT(act, wd)


def _body(x_ref, wr_ref, wu_ref, wg_ref, wd_ref, wus_ref, wgs_ref, wds_ref,
          out_ref, scale_ref):
    h = pl.program_id(0)
    j = pl.program_id(1)

    @pl.when((h == 0) & (j == 0))
    def _init():
        logits = _dotT(x_ref[...], wr_ref[...])
        m = jnp.max(logits, axis=-1, keepdims=True)
        e = jnp.exp(logits - m)
        p = e / jnp.sum(e, axis=-1, keepdims=True)
        amax = jnp.argmax(logits, axis=-1)
        pmax = jnp.max(p, axis=-1)
        cols = jax.lax.broadcasted_iota(jnp.int32, (TOKENS, NUM_EXPERTS), 1)
        scale = jnp.where(cols == amax[:, None], ALPHA * pmax[:, None], 0.0)
        scale_ref[...] = jnp.where(cols == N_ROUTED, 1.0, scale)
        out_ref[...] = jnp.zeros_like(out_ref)

    x = x_ref[...]
    cols = jax.lax.broadcasted_iota(jnp.int32, (TOKENS, NUM_EXPERTS), 1)
    s = jnp.sum(jnp.where(cols == j, scale_ref[...], 0.0), axis=1,
                keepdims=True)

    @pl.when(j < N_ROUTED)
    def _routed():
        out_ref[...] += _swiglu_part(x, wu_ref[0], wg_ref[0], wd_ref[0]) * s

    @pl.when(j == N_ROUTED)
    def _shared():
        out_ref[...] += _swiglu_part(x, wus_ref[...], wgs_ref[...],
                                     wds_ref[...]) * s


@jax.jit
def kernel(x, Wr, Wu, Wg, Wd, Wu_s, Wg_s, Wd_s):
    jc = lambda j: jnp.minimum(j, N_ROUTED - 1)
    return pl.pallas_call(
        _body,
        grid=(NH, NUM_EXPERTS),
        in_specs=[
            pl.BlockSpec((TOKENS, DIM), lambda h, j: (0, 0)),
            pl.BlockSpec((NUM_EXPERTS, DIM), lambda h, j: (0, 0)),
            pl.BlockSpec((1, HB, DIM), lambda h, j: (jc(j), h, 0)),
            pl.BlockSpec((1, HB, DIM), lambda h, j: (jc(j), h, 0)),
            pl.BlockSpec((1, DIM, HB), lambda h, j: (jc(j), 0, h)),
            pl.BlockSpec((HB, DIM), lambda h, j: (h, 0)),
            pl.BlockSpec((HB, DIM), lambda h, j: (h, 0)),
            pl.BlockSpec((DIM, HB), lambda h, j: (0, h)),
        ],
        out_specs=pl.BlockSpec((TOKENS, DIM), lambda h, j: (0, 0)),
        out_shape=jax.ShapeDtypeStruct((TOKENS, DIM), jnp.float32),
        scratch_shapes=[pltpu.VMEM((TOKENS, NUM_EXPERTS), jnp.float32)],
        compiler_params=pltpu.CompilerParams(
            dimension_semantics=("arbitrary", "arbitrary"),
        ),
    )(x, Wr, Wu, Wg, Wd, Wu_s, Wg_s, Wd_s)
